# baseline (speedup 1.0000x reference)
import functools

import jax
import jax.numpy as jnp
from jax import lax
from jax.experimental import pallas as pl
from jax.experimental.pallas import tpu as pltpu
from jax.experimental.pallas import tpu_sc as plsc

SEQ = 1024
NC, NS = 2, 16
NW = NC * NS
TOK_PER_TILE_B = 128
TOK_PER_CHUNK = 8
K = 14
CH = TOK_PER_CHUNK * K
NCH = TOK_PER_TILE_B // TOK_PER_CHUNK


def _matmul_block(emb_ref, w_ref, b_ref, out_ref):
    acc = jnp.dot(emb_ref[...], w_ref[...], preferred_element_type=jnp.float32)
    sel = (pl.program_id(0) == 0).astype(jnp.float32)
    out_ref[...] = (acc + sel * b_ref[...]).astype(jnp.bfloat16)


def _stage_a(emb_p, W, b2):
    VP, H = emb_p.shape
    O = W.shape[1]
    nrb = VP // 512
    return pl.pallas_call(
        _matmul_block,
        grid=(3, nrb),
        in_specs=[
            pl.BlockSpec((512, H), lambda k, i: (i, 0)),
            pl.BlockSpec((512, O), lambda k, i: (k, 0)),
            pl.BlockSpec((1, O), lambda k, i: (0, 0)),
        ],
        out_specs=pl.BlockSpec((512, O), lambda k, i: (k * nrb + i, 0)),
        out_shape=jax.ShapeDtypeStruct((3 * VP, O), jnp.bfloat16),
    )(emb_p, W, b2)


def _stage_b(tcat, idx3, U, O):
    NG = O // 32

    @functools.partial(
        pl.kernel,
        mesh=plsc.VectorSubcoreMesh(core_axis_name="c", subcore_axis_name="s"),
        out_type=jax.ShapeDtypeStruct((U, O), jnp.float32),
        scratch_types=[
            pltpu.VMEM((NCH, CH), jnp.int32),
            pltpu.VMEM((2, CH, O // 2), jnp.int32),
            pltpu.VMEM((2, TOK_PER_CHUNK, O), jnp.float32),
            pltpu.SemaphoreType.DMA,
            pltpu.SemaphoreType.DMA,
            pltpu.SemaphoreType.DMA,
        ],
    )
    def body(tcat_hbm, idx_hbm, out_hbm, idx_v, stage_v, outst_v, gsem, ws0, ws1):
        cid = lax.axis_index("c")
        sid = lax.axis_index("s")
        wid = cid * NS + sid
        row0 = wid * TOK_PER_TILE_B
        pltpu.sync_copy(idx_hbm.at[wid], idx_v)

        wsems = (ws0, ws1)
        gathers = [None, None]
        writes = [None, None]
        gathers[0] = pltpu.async_copy(
            tcat_hbm.at[idx_v.at[0]], stage_v.at[0], gsem
        )
        for j in range(NCH):
            p = j % 2
            gathers[p].wait()
            if j + 1 < NCH:
                gathers[1 - p] = pltpu.async_copy(
                    tcat_hbm.at[idx_v.at[j + 1]], stage_v.at[1 - p], gsem
                )
            if writes[p] is not None:
                writes[p].wait()

            hi_mask = jnp.full((16,), -65536, dtype=jnp.int32)

            def expand(r, w0):
                v = stage_v[p, r, pl.ds(w0, 16)]
                lo = lax.bitcast_convert_type(v << 16, jnp.float32)
                hi = lax.bitcast_convert_type(v & hi_mask, jnp.float32)
                return lo, hi

            def red(g, _):
                w0 = g * 16
                c0 = g * 32
                for t in range(TOK_PER_CHUNK):
                    va, vb = expand(t * K, w0)
                    for k in range(1, K):
                        a, b = expand(t * K + k, w0)
                        va = va + a
                        vb = vb + b
                    outst_v[p, t, pl.ds(c0, 16)] = va
                    outst_v[p, t, pl.ds(c0 + 16, 16)] = vb
                return 0

            lax.fori_loop(0, NG, red, 0)
            writes[p] = pltpu.async_copy(
                outst_v.at[p],
                out_hbm.at[pl.ds(row0 + j * TOK_PER_CHUNK, TOK_PER_CHUNK)],
                wsems[p],
            )
        for w in writes:
            if w is not None:
                w.wait()

    return body(tcat, idx3)


def _stage_c(out_u, inv3, T, O):
    tok_per_tile = T // NW
    nchunk = tok_per_tile // 128
    bsz = T // SEQ
    tiles_per_seq = SEQ // tok_per_tile

    @functools.partial(
        pl.kernel,
        mesh=plsc.VectorSubcoreMesh(core_axis_name="c", subcore_axis_name="s"),
        compiler_params=pltpu.CompilerParams(use_tc_tiling_on_sc=False),
        out_type=jax.ShapeDtypeStruct((bsz * (SEQ + 2), O), jnp.float32),
        scratch_types=[
            pltpu.VMEM((nchunk, 128), jnp.int32),
            pltpu.VMEM((128, O), jnp.float32),
            pltpu.SemaphoreType.DMA,
        ],
    )
    def body(src_hbm, inv_hbm, out_hbm, idx_v, stage_v, sem):
        cid = lax.axis_index("c")
        sid = lax.axis_index("s")
        wid = cid * NS + sid
        seq = wid // tiles_per_seq
        lane = wid % tiles_per_seq
        dst_base = seq * (SEQ + 2) + 1 + lane * tok_per_tile
        pltpu.sync_copy(inv_hbm.at[wid], idx_v)
        for j in range(nchunk):
            pltpu.async_copy(src_hbm.at[idx_v.at[j]], stage_v, sem).wait()
            pltpu.sync_copy(stage_v, out_hbm.at[pl.ds(dst_base + j * 128, 128)])
        zv = jnp.zeros((16,), jnp.float32)

        def zr(i, _):
            stage_v[0, pl.ds(i * 16, 16)] = zv
            return 0

        lax.fori_loop(0, O // 16, zr, 0)

        @pl.when(lane == 0)
        def _():
            pltpu.sync_copy(stage_v.at[pl.ds(0, 1)],
                            out_hbm.at[pl.ds(seq * (SEQ + 2), 1)])

        @pl.when(lane == tiles_per_seq - 1)
        def _():
            pltpu.sync_copy(stage_v.at[pl.ds(0, 1)],
                            out_hbm.at[pl.ds(seq * (SEQ + 2) + SEQ + 1, 1)])

    return body(out_u, inv3)


def kernel(first, mid, last, inv_i, seq_len, emb_table, W, b):
    V, H = emb_table.shape
    O = W.shape[1]
    U = first.shape[0]
    T = inv_i.shape[0]
    bsz = T // SEQ
    VP = 4096

    emb_p = jnp.pad(emb_table, ((0, VP - V), (0, 0)))
    g = jnp.arange(O, dtype=jnp.int32)
    grp, off = g // 32, g % 32
    perm = grp * 32 + jnp.where(off % 2 == 0, off // 2, 16 + off // 2)
    tcat = _stage_a(emb_p, W[:, perm], b[perm].reshape(1, O))
    tcat = jax.lax.bitcast_convert_type(
        tcat.reshape(3 * VP, O // 2, 2), jnp.int32
    )

    first = first.astype(jnp.int32)
    mid = mid.astype(jnp.int32)
    last = last.astype(jnp.int32)
    inv_i = inv_i.astype(jnp.int32)

    idx_all = jnp.concatenate(
        [first[:, None], mid + VP, last[:, None] + 2 * VP], axis=1
    )
    idx3 = idx_all.reshape(NW, NCH, CH)
    out_u = _stage_b(tcat, idx3, U, O)

    inv3 = inv_i.reshape(NW, T // NW // 128, 128)
    flat = _stage_c(out_u, inv3, T, O)
    return flat.reshape(bsz, SEQ + 2, O)


# speedup vs baseline: 2.0610x; 2.0610x over previous
import functools

import jax
import jax.numpy as jnp
from jax import lax
from jax.experimental import pallas as pl
from jax.experimental.pallas import tpu as pltpu
from jax.experimental.pallas import tpu_sc as plsc

SEQ = 1024
NC, NS = 2, 16
NW = NC * NS
TOK_PER_TILE_B = 128
TOK_PER_CHUNK = 8
K = 14
CH = TOK_PER_CHUNK * K
NCH = TOK_PER_TILE_B // TOK_PER_CHUNK


def _rne_bf16_bits(x):
    u = jax.lax.bitcast_convert_type(x, jnp.uint32)
    lsb = (u >> 16) & jnp.uint32(1)
    return (u + jnp.uint32(0x7FFF) + lsb) >> 16


def _matmul_block(emb_ref, w_ref, b_ref, out_ref):
    lhs = emb_ref[...].astype(jnp.bfloat16)
    rhs = w_ref[...].astype(jnp.bfloat16)
    acc = jnp.dot(lhs, rhs, preferred_element_type=jnp.float32)
    sel = (pl.program_id(0) == 0).astype(jnp.float32)
    acc = acc + sel * b_ref[...]
    h = acc.shape[1] // 2
    lo = _rne_bf16_bits(acc[:, :h])
    hi = _rne_bf16_bits(acc[:, h:])
    out_ref[...] = jax.lax.bitcast_convert_type((hi << 16) | lo, jnp.int32)


def _stage_a(emb, W, b2):
    V, H = emb.shape
    O = W.shape[1]
    nrb = 4
    rb = V // nrb
    return pl.pallas_call(
        _matmul_block,
        grid=(3, nrb),
        in_specs=[
            pl.BlockSpec((rb, H), lambda k, i: (i, 0)),
            pl.BlockSpec((H, O), lambda k, i: (k, 0)),
            pl.BlockSpec((1, O), lambda k, i: (0, 0)),
        ],
        out_specs=pl.BlockSpec((rb, O // 2), lambda k, i: (k * nrb + i, 0)),
        out_shape=jax.ShapeDtypeStruct((3 * V, O // 2), jnp.int32),
    )(emb, W, b2)


def _stage_b(tcat, idx3, U, O):
    NG = O // 32

    @functools.partial(
        pl.kernel,
        mesh=plsc.VectorSubcoreMesh(core_axis_name="c", subcore_axis_name="s"),
        out_type=jax.ShapeDtypeStruct((U + 8, O), jnp.float32),
        scratch_types=[
            pltpu.VMEM((NCH, CH), jnp.int32),
            pltpu.VMEM((2, CH, O // 2), jnp.int32),
            pltpu.VMEM((2, TOK_PER_CHUNK, O), jnp.float32),
            pltpu.SemaphoreType.DMA,
            pltpu.SemaphoreType.DMA,
            pltpu.SemaphoreType.DMA,
        ],
    )
    def body(tcat_hbm, idx_hbm, out_hbm, idx_v, stage_v, outst_v, gsem, ws0, ws1):
        cid = lax.axis_index("c")
        sid = lax.axis_index("s")
        wid = cid * NS + sid
        row0 = wid * TOK_PER_TILE_B
        pltpu.sync_copy(idx_hbm.at[wid], idx_v)

        wsems = (ws0, ws1)
        gathers = [None, None]
        writes = [None, None]
        gathers[0] = pltpu.async_copy(
            tcat_hbm.at[idx_v.at[0]], stage_v.at[0], gsem
        )
        for j in range(NCH):
            p = j % 2
            gathers[p].wait()
            if j + 1 < NCH:
                gathers[1 - p] = pltpu.async_copy(
                    tcat_hbm.at[idx_v.at[j + 1]], stage_v.at[1 - p], gsem
                )
            if writes[p] is not None:
                writes[p].wait()

            hi_mask = jnp.full((16,), -65536, dtype=jnp.int32)

            def expand(r, w0):
                v = stage_v[p, r, pl.ds(w0, 16)]
                lo = lax.bitcast_convert_type(v << 16, jnp.float32)
                hi = lax.bitcast_convert_type(v & hi_mask, jnp.float32)
                return lo, hi

            def red(g, _):
                w0 = g * 16
                c0 = g * 32
                for t in range(TOK_PER_CHUNK):
                    va, vb = expand(t * K, w0)
                    for k in range(1, K):
                        a, b = expand(t * K + k, w0)
                        va = va + a
                        vb = vb + b
                    outst_v[p, t, pl.ds(c0, 16)] = va
                    outst_v[p, t, pl.ds(c0 + 16, 16)] = vb
                return 0

            lax.fori_loop(0, NG, red, 0)
            writes[p] = pltpu.async_copy(
                outst_v.at[p],
                out_hbm.at[pl.ds(row0 + j * TOK_PER_CHUNK, TOK_PER_CHUNK)],
                wsems[p],
            )
        for w in writes:
            if w is not None:
                w.wait()

        @pl.when(wid == 0)
        def _():
            zv = jnp.zeros((16,), jnp.float32)

            def zb(i, _):
                outst_v[0, i // (O // 16), pl.ds((i % (O // 16)) * 16, 16)] = zv
                return 0

            lax.fori_loop(0, TOK_PER_CHUNK * (O // 16), zb, 0)
            pltpu.sync_copy(outst_v.at[0], out_hbm.at[pl.ds(U, 8)])

    return body(tcat, idx3)


def _stage_c(out_ue, idxc, nrows, O):
    CC = 64
    per_tile = 4

    @functools.partial(
        pl.kernel,
        mesh=plsc.VectorSubcoreMesh(core_axis_name="c", subcore_axis_name="s"),
        out_type=jax.ShapeDtypeStruct((nrows, O), jnp.float32),
        scratch_types=[
            pltpu.VMEM((per_tile + 1, CC), jnp.int32),
            pltpu.VMEM((2, CC, O), jnp.float32),
            pltpu.SemaphoreType.DMA,
            pltpu.SemaphoreType.DMA,
            pltpu.SemaphoreType.DMA,
        ],
    )
    def body(src_hbm, idx_hbm, out_hbm, idx_v, stage_v, gsem, ws0, ws1):
        cid = lax.axis_index("c")
        sid = lax.axis_index("s")
        wid = cid * NS + sid
        c0 = wid * per_tile
        pltpu.sync_copy(idx_hbm.at[wid], idx_v)
        wsems = (ws0, ws1)
        gathers = [None, None]
        writes = [None, None]
        gathers[0] = pltpu.async_copy(src_hbm.at[idx_v.at[0]], stage_v.at[0], gsem)
        for j in range(per_tile):
            p = j % 2
            gathers[p].wait()
            if j + 1 < per_tile:
                if writes[1 - p] is not None:
                    writes[1 - p].wait()
                gathers[1 - p] = pltpu.async_copy(
                    src_hbm.at[idx_v.at[j + 1]], stage_v.at[1 - p], gsem
                )
            writes[p] = pltpu.async_copy(
                stage_v.at[p], out_hbm.at[pl.ds((c0 + j) * CC, CC)], wsems[p]
            )
        for w in writes:
            if w is not None:
                w.wait()

        tail_rows = nrows - NW * per_tile * CC
        if tail_rows > 0:
            @pl.when(wid == 0)
            def _():
                pltpu.async_copy(
                    src_hbm.at[idx_v.at[per_tile]], stage_v.at[0], gsem
                ).wait()
                pltpu.sync_copy(
                    stage_v.at[0].at[pl.ds(0, tail_rows)],
                    out_hbm.at[pl.ds(NW * per_tile * CC, tail_rows)],
                )

    return body(out_ue, idxc)


def kernel(first, mid, last, inv_i, seq_len, emb_table, W, b):
    V, H = emb_table.shape
    O = W.shape[1]
    U = first.shape[0]
    T = inv_i.shape[0]
    bsz = T // SEQ

    m = jnp.arange(O // 2, dtype=jnp.int32)
    half = (m // 16) * 32 + (m % 16)
    perm = jnp.concatenate([half, half + 16])
    tcat = _stage_a(emb_table, W[:, perm], b[perm].reshape(1, O))

    first = first.astype(jnp.int32)
    mid = mid.astype(jnp.int32)
    last = last.astype(jnp.int32)
    inv_i = inv_i.astype(jnp.int32)

    idx_all = jnp.concatenate(
        [first[:, None], mid + V, last[:, None] + 2 * V], axis=1
    )
    idx3 = idx_all.reshape(NW, NCH, CH)
    out_ue = _stage_b(tcat, idx3, U, O)

    nrows = bsz * (SEQ + 2)
    zcol = jnp.full((bsz, 1), U, dtype=jnp.int32)
    idxc = jnp.concatenate([zcol, inv_i.reshape(bsz, SEQ), zcol], axis=1)
    idxc = idxc.reshape(nrows)
    main = idxc[: NW * 4 * 64].reshape(NW, 4, 64)
    tail = jnp.full((NW, 1, 64), U, dtype=jnp.int32)
    tail = tail.at[0, 0, : nrows - NW * 4 * 64].set(idxc[NW * 4 * 64 :])
    idxc5 = jnp.concatenate([main, tail], axis=1)
    flat = _stage_c(out_ue, idxc5, nrows, O)
    return flat.reshape(bsz, SEQ + 2, O)
